# baseline (speedup 1.0000x reference)
import functools

import jax
import jax.numpy as jnp
from jax import lax
from jax.experimental import pallas as pl
from jax.experimental.pallas import tpu as pltpu
from jax.experimental.pallas import tpu_sc as plsc

_NC = 2
_NS = 16
_NW = _NC * _NS
_CHUNK = 128
_LANES = 16


def _round_up(v, m):
    return ((v + m - 1) // m) * m


@functools.lru_cache(maxsize=None)
def _make_deg_kernel(cpw, n_rows):
    tr = n_rows // _NS
    mesh = plsc.VectorSubcoreMesh(
        core_axis_name="c", subcore_axis_name="s", num_cores=_NC,
        num_subcores=_NS)

    @functools.partial(
        pl.kernel,
        out_type=jax.ShapeDtypeStruct((_NC * n_rows,), jnp.float32),
        mesh=mesh,
        scratch_types=[
            pltpu.VMEM((cpw, _CHUNK), jnp.int32),
            pltpu.VMEM((_CHUNK,), jnp.float32),
            pltpu.VMEM((tr,), jnp.float32),
            pltpu.VMEM_SHARED((n_rows,), jnp.float32),
        ],
    )
    def deg_kernel(dst_hbm, out_hbm, dst_v, ones_v, zero_v, deg_sh):
        c = lax.axis_index("c")
        s = lax.axis_index("s")
        wid = s * _NC + c
        for i in range(_CHUNK // _LANES):
            ones_v[pl.ds(i * _LANES, _LANES)] = jnp.ones((_LANES,),
                                                         jnp.float32)

        def zstep(i, carry):
            zero_v[pl.ds(i * _LANES, _LANES)] = jnp.zeros((_LANES,),
                                                          jnp.float32)
            return carry

        lax.fori_loop(0, tr // _LANES, zstep, 0)
        pltpu.sync_copy(zero_v, deg_sh.at[pl.ds(s * tr, tr)])
        pltpu.sync_copy(dst_hbm.at[wid], dst_v)
        plsc.subcore_barrier()

        def step(j, carry):
            pltpu.sync_copy(ones_v, deg_sh.at[dst_v.at[j]], add=True)
            return carry

        lax.fori_loop(0, cpw, step, 0)
        plsc.subcore_barrier()
        pltpu.sync_copy(deg_sh.at[pl.ds(s * tr, tr)],
                        out_hbm.at[pl.ds(c * n_rows + s * tr, tr)])

    return deg_kernel


@functools.lru_cache(maxsize=None)
def _make_scatter_kernel(cpw, n_rows, d):
    tr = n_rows // _NS
    mesh = plsc.VectorSubcoreMesh(
        core_axis_name="c", subcore_axis_name="s", num_cores=_NC,
        num_subcores=_NS)

    hpw = cpw // 2

    @functools.partial(
        pl.kernel,
        out_type=jax.ShapeDtypeStruct((_NC, n_rows, d), jnp.float32),
        mesh=mesh,
        scratch_types=[
            pltpu.VMEM((hpw, _CHUNK), jnp.int32),
            pltpu.VMEM((hpw, _CHUNK), jnp.int32),
            pltpu.VMEM((_CHUNK, d), jnp.float32),
            pltpu.VMEM((_CHUNK, d), jnp.float32),
            pltpu.VMEM_SHARED((n_rows, d), jnp.float32),
            pltpu.SemaphoreType.DMA,
            pltpu.SemaphoreType.DMA,
        ],
    )
    def scat_kernel(u_hbm, src_hbm, dst_hbm, out_hbm, src_v, dst_v, rows0,
                    rows1, acc_sh, sem0, sem1):
        c = lax.axis_index("c")
        s = lax.axis_index("s")
        wid = s * _NC + c

        def zrow(i, carry):
            for j in range(d // _LANES):
                rows0[i, pl.ds(j * _LANES, _LANES)] = jnp.zeros(
                    (_LANES,), jnp.float32)
            return carry

        lax.fori_loop(0, _CHUNK, zrow, 0)
        for k in range(tr // _CHUNK):
            pltpu.sync_copy(
                rows0, acc_sh.at[pl.ds(s * tr + k * _CHUNK, _CHUNK)])
        plsc.subcore_barrier()

        for p in range(2):
            pltpu.sync_copy(src_hbm.at[wid, pl.ds(p * hpw, hpw)], src_v)
            pltpu.sync_copy(dst_hbm.at[wid, pl.ds(p * hpw, hpw)], dst_v)

            def step(j, carry):
                pltpu.async_copy(u_hbm.at[src_v.at[j]], rows0, sem0).wait()
                pltpu.sync_copy(rows0, acc_sh.at[dst_v.at[j]], add=True)
                return carry

            lax.fori_loop(0, hpw, step, 0)
        plsc.subcore_barrier()
        pltpu.sync_copy(acc_sh.at[pl.ds(s * tr, tr)],
                        out_hbm.at[c, pl.ds(s * tr, tr)])

    return scat_kernel


def _dinv_col(degp, n):
    deg = degp[0:1, :] + degp[1:2, :] + 1.0
    return jnp.transpose(lax.rsqrt(deg)[:, :n])


def _tc1_body(x_ref, w_ref, degp_ref, u_ref):
    n = x_ref.shape[0]
    dinv = _dinv_col(degp_ref[...], n)
    u_ref[...] = jnp.dot(x_ref[...], w_ref[...],
                         preferred_element_type=jnp.float32, precision=lax.Precision.HIGHEST) * dinv


def _bn_relu(t, gamma, beta, eps=1e-5):
    mu = jnp.mean(t, axis=0, keepdims=True)
    var = jnp.mean((t - mu) ** 2, axis=0, keepdims=True)
    return jax.nn.relu(gamma * (t - mu) * lax.rsqrt(var + eps) + beta)


def _tc2_body(acc_ref, u_ref, degp_ref, bc_ref, g_ref, be_ref, w_ref,
              out_ref):
    n = u_ref.shape[0]
    dinv = _dinv_col(degp_ref[...], n)
    t = (acc_ref[0, :n, :] + acc_ref[1, :n, :] + u_ref[...]) * dinv \
        + bc_ref[...]
    y = _bn_relu(t, g_ref[...], be_ref[...])
    out_ref[...] = jnp.dot(y, w_ref[...],
                           preferred_element_type=jnp.float32, precision=lax.Precision.HIGHEST) * dinv


def _tc3_body(acc_ref, u_ref, degp_ref, bc_ref, g_ref, be_ref, batch_ref,
              w1_ref, b1_ref, w2_ref, b2_ref, w3_ref, b3_ref, out_ref):
    n = u_ref.shape[0]
    g = out_ref.shape[0]
    dinv = _dinv_col(degp_ref[...], n)
    t = (acc_ref[0, :n, :] + acc_ref[1, :n, :] + u_ref[...]) * dinv \
        + bc_ref[...]
    h = _bn_relu(t, g_ref[...], be_ref[...])
    seg = lax.broadcasted_iota(jnp.int32, (g, n), 0)
    onehot_t = (seg == batch_ref[...]).astype(jnp.float32)
    pooled = jnp.dot(onehot_t, h, preferred_element_type=jnp.float32, precision=lax.Precision.HIGHEST)
    z = jax.nn.relu(jnp.dot(pooled, w1_ref[...],
                            preferred_element_type=jnp.float32, precision=lax.Precision.HIGHEST) + b1_ref[...])
    z = jax.nn.relu(jnp.dot(z, w2_ref[...],
                            preferred_element_type=jnp.float32, precision=lax.Precision.HIGHEST) + b2_ref[...])
    out_ref[...] = jnp.dot(z, w3_ref[...],
                           preferred_element_type=jnp.float32, precision=lax.Precision.HIGHEST) + b3_ref[...]


def kernel(x, edge_index, batch, Wc1, bc1, g1, be1, Wc2, bc2, g2, be2,
           W1, b1, W2, b2, W3, b3):
    n, d = x.shape
    e = edge_index.shape[1]
    g = 128

    cpw = _round_up(-(-e // (_NW * _CHUNK)), 16)
    e_pad = _NW * cpw * _CHUNK
    pad = e_pad - e
    n_rows = _round_up(n + 1, _NS * _CHUNK)

    src = jnp.concatenate(
        [edge_index[0], jnp.zeros((pad,), edge_index.dtype)])
    dst = jnp.concatenate(
        [edge_index[1], jnp.full((pad,), n, edge_index.dtype)])
    src2 = src.reshape(_NW, cpw, _CHUNK)
    dst2 = dst.reshape(_NW, cpw, _CHUNK)
    batch_row = batch.reshape(1, n)

    degp = _make_deg_kernel(cpw, n_rows)(dst2).reshape(_NC, n_rows)

    u1 = pl.pallas_call(
        _tc1_body,
        out_shape=jax.ShapeDtypeStruct((n, Wc1.shape[1]), jnp.float32),
    )(x, Wc1, degp)

    scat = _make_scatter_kernel(cpw, n_rows, d)
    acc1 = scat(u1, src2, dst2)

    u2 = pl.pallas_call(
        _tc2_body,
        out_shape=jax.ShapeDtypeStruct((n, Wc2.shape[1]), jnp.float32),
    )(acc1, u1, degp, bc1, g1, be1, Wc2)

    acc2 = scat(u2, src2, dst2)

    out = pl.pallas_call(
        _tc3_body,
        out_shape=jax.ShapeDtypeStruct((g, W3.shape[1]), jnp.float32),
    )(acc2, u2, degp, bc2, g2, be2, batch_row, W1, b1, W2, b2, W3, b3)
    return out


# speedup vs baseline: 1.5515x; 1.5515x over previous
import functools

import jax
import jax.numpy as jnp
from jax import lax
from jax.experimental import pallas as pl
from jax.experimental.pallas import tpu as pltpu
from jax.experimental.pallas import tpu_sc as plsc

_NC = 2
_NS = 16
_NW = _NC * _NS
_CHUNK = 128
_LANES = 16


def _round_up(v, m):
    return ((v + m - 1) // m) * m


@functools.lru_cache(maxsize=None)
def _make_deg_kernel(cpw, n_rows):
    tr = n_rows // _NS
    mesh = plsc.VectorSubcoreMesh(
        core_axis_name="c", subcore_axis_name="s", num_cores=_NC,
        num_subcores=_NS)

    @functools.partial(
        pl.kernel,
        out_type=jax.ShapeDtypeStruct((_NC * n_rows,), jnp.float32),
        mesh=mesh,
        scratch_types=[
            pltpu.VMEM((cpw, _CHUNK), jnp.int32),
            pltpu.VMEM((_CHUNK,), jnp.float32),
            pltpu.VMEM((tr,), jnp.float32),
            pltpu.VMEM_SHARED((n_rows,), jnp.float32),
        ],
    )
    def deg_kernel(dst_hbm, out_hbm, dst_v, ones_v, zero_v, deg_sh):
        c = lax.axis_index("c")
        s = lax.axis_index("s")
        wid = s * _NC + c
        for i in range(_CHUNK // _LANES):
            ones_v[pl.ds(i * _LANES, _LANES)] = jnp.ones((_LANES,),
                                                         jnp.float32)

        def zstep(i, carry):
            zero_v[pl.ds(i * _LANES, _LANES)] = jnp.zeros((_LANES,),
                                                          jnp.float32)
            return carry

        lax.fori_loop(0, tr // _LANES, zstep, 0)
        pltpu.sync_copy(zero_v, deg_sh.at[pl.ds(s * tr, tr)])
        pltpu.sync_copy(dst_hbm.at[wid], dst_v)
        plsc.subcore_barrier()

        def step(j, carry):
            pltpu.sync_copy(ones_v, deg_sh.at[dst_v.at[j]], add=True)
            return carry

        lax.fori_loop(0, cpw, step, 0)
        plsc.subcore_barrier()
        pltpu.sync_copy(deg_sh.at[pl.ds(s * tr, tr)],
                        out_hbm.at[pl.ds(c * n_rows + s * tr, tr)])

    return deg_kernel


@functools.lru_cache(maxsize=None)
def _make_scatter_kernel(cpw, n_rows, d):
    tr = n_rows // _NS
    mesh = plsc.VectorSubcoreMesh(
        core_axis_name="c", subcore_axis_name="s", num_cores=_NC,
        num_subcores=_NS)

    @functools.partial(
        pl.kernel,
        out_type=jax.ShapeDtypeStruct((_NC, n_rows, d), jnp.float32),
        mesh=mesh,
        scratch_types=[
            pltpu.VMEM((cpw, _CHUNK), jnp.int32),
            pltpu.VMEM((cpw, _CHUNK), jnp.int32),
            pltpu.VMEM((_CHUNK, d), jnp.float32),
            pltpu.VMEM_SHARED((n_rows, d), jnp.float32),
            pltpu.SemaphoreType.DMA,
        ],
    )
    def scat_kernel(u_hbm, src_hbm, dst_hbm, out_hbm, src_v, dst_v, rows_v,
                    acc_sh, sem):
        c = lax.axis_index("c")
        s = lax.axis_index("s")
        wid = s * _NC + c

        def zrow(i, carry):
            for j in range(d // _LANES):
                rows_v[i, pl.ds(j * _LANES, _LANES)] = jnp.zeros(
                    (_LANES,), jnp.float32)
            return carry

        lax.fori_loop(0, _CHUNK, zrow, 0)
        for k in range(tr // _CHUNK):
            pltpu.sync_copy(
                rows_v, acc_sh.at[pl.ds(s * tr + k * _CHUNK, _CHUNK)])
        pltpu.sync_copy(src_hbm.at[wid], src_v)
        pltpu.sync_copy(dst_hbm.at[wid], dst_v)
        plsc.subcore_barrier()

        def step(j, carry):
            pltpu.async_copy(u_hbm.at[src_v.at[j]], rows_v, sem).wait()
            pltpu.sync_copy(rows_v, acc_sh.at[dst_v.at[j]], add=True)
            return carry

        lax.fori_loop(0, cpw, step, 0)
        plsc.subcore_barrier()
        pltpu.sync_copy(acc_sh.at[pl.ds(s * tr, tr)],
                        out_hbm.at[c, pl.ds(s * tr, tr)])

    return scat_kernel


def _dot_ref(a, b):
    return jnp.dot(a.astype(jnp.bfloat16), b.astype(jnp.bfloat16),
                   preferred_element_type=jnp.float32)


def _dinv_col(degp, n):
    deg = degp[0:1, :] + degp[1:2, :] + 1.0
    return jnp.transpose((1.0 / jnp.sqrt(deg))[:, :n])


def _tc1_body(x_ref, w_ref, degp_ref, u_ref):
    n = x_ref.shape[0]
    dinv = _dinv_col(degp_ref[...], n)
    u_ref[...] = _dot_ref(x_ref[...], w_ref[...]) * dinv


def _bn_relu(t, gamma, beta, eps=1e-5):
    mu = jnp.mean(t, axis=0, keepdims=True)
    var = jnp.mean((t - mu) ** 2, axis=0, keepdims=True)
    return jax.nn.relu(gamma * (t - mu) / jnp.sqrt(var + eps) + beta)


def _tc2_body(acc_ref, u_ref, degp_ref, bc_ref, g_ref, be_ref, w_ref,
              out_ref):
    n = u_ref.shape[0]
    dinv = _dinv_col(degp_ref[...], n)
    t = (acc_ref[0, :n, :] + acc_ref[1, :n, :] + u_ref[...]) * dinv \
        + bc_ref[...]
    y = _bn_relu(t, g_ref[...], be_ref[...])
    out_ref[...] = _dot_ref(y, w_ref[...]) * dinv


def _tc3_body(acc_ref, u_ref, degp_ref, bc_ref, g_ref, be_ref, batch_ref,
              w1_ref, b1_ref, w2_ref, b2_ref, w3_ref, b3_ref, out_ref):
    n = u_ref.shape[0]
    g = out_ref.shape[0]
    dinv = _dinv_col(degp_ref[...], n)
    t = (acc_ref[0, :n, :] + acc_ref[1, :n, :] + u_ref[...]) * dinv \
        + bc_ref[...]
    h = _bn_relu(t, g_ref[...], be_ref[...])
    seg = lax.broadcasted_iota(jnp.int32, (g, n), 0)
    onehot_t = (seg == batch_ref[...]).astype(jnp.float32)
    pooled = jnp.dot(onehot_t, h, preferred_element_type=jnp.float32,
                     precision=lax.Precision.HIGHEST)
    z = jax.nn.relu(_dot_ref(pooled, w1_ref[...]) + b1_ref[...])
    z = jax.nn.relu(_dot_ref(z, w2_ref[...]) + b2_ref[...])
    out_ref[...] = _dot_ref(z, w3_ref[...]) + b3_ref[...]


def kernel(x, edge_index, batch, Wc1, bc1, g1, be1, Wc2, bc2, g2, be2,
           W1, b1, W2, b2, W3, b3):
    n, d = x.shape
    e = edge_index.shape[1]
    g = 128

    cpw = -(-e // (_NW * _CHUNK))
    e_pad = _NW * cpw * _CHUNK
    pad = e_pad - e
    n_rows = _round_up(n + 1, _NS * _CHUNK)

    src = jnp.concatenate(
        [edge_index[0], jnp.zeros((pad,), edge_index.dtype)])
    dst = jnp.concatenate(
        [edge_index[1], jnp.full((pad,), n, edge_index.dtype)])
    src2 = src.reshape(_NW, cpw, _CHUNK)
    dst2 = dst.reshape(_NW, cpw, _CHUNK)
    batch_row = batch.reshape(1, n)

    degp = _make_deg_kernel(cpw, n_rows)(dst2).reshape(_NC, n_rows)

    u1 = pl.pallas_call(
        _tc1_body,
        out_shape=jax.ShapeDtypeStruct((n, Wc1.shape[1]), jnp.float32),
    )(x, Wc1, degp)

    scat = _make_scatter_kernel(cpw, n_rows, d)
    acc1 = scat(u1, src2, dst2)

    u2 = pl.pallas_call(
        _tc2_body,
        out_shape=jax.ShapeDtypeStruct((n, Wc2.shape[1]), jnp.float32),
    )(acc1, u1, degp, bc1, g1, be1, Wc2)

    acc2 = scat(u2, src2, dst2)

    out = pl.pallas_call(
        _tc3_body,
        out_shape=jax.ShapeDtypeStruct((g, W3.shape[1]), jnp.float32),
    )(acc2, u2, degp, bc2, g2, be2, batch_row, W1, b1, W2, b2, W3, b3)
    return out
